# baseline (speedup 1.0000x reference)
import jax
import jax.numpy as jnp
from jax.experimental import pallas as pl
from jax.experimental.pallas import tpu as pltpu

_BN_EPS = 1e-5


def _make_conv_stats_kernel(B, H, W):
    def _conv_stats_kernel(x_ref, w_ref, conv_ref, stats_ref):
        _, C, M = x_ref.shape

        m = jax.lax.broadcasted_iota(jnp.int32, (1, M), 1)
        col = jax.lax.rem(m, W)
        row_ok = {-1: m >= W, 0: None, 1: m < M - W}
        col_ok = {-1: col > 0, 0: None, 1: col < W - 1}

        for b in range(B):
            x = x_ref[b].astype(jnp.bfloat16)
            parts = []
            for ky in range(3):
                dy = ky - 1
                for kx in range(3):
                    dx = kx - 1
                    s = dy * W + dx
                    t = x if s == 0 else jnp.roll(x, -s, axis=1)
                    mask = None
                    if row_ok[dy] is not None and col_ok[dx] is not None:
                        mask = jnp.logical_and(row_ok[dy], col_ok[dx])
                    elif row_ok[dy] is not None:
                        mask = row_ok[dy]
                    elif col_ok[dx] is not None:
                        mask = col_ok[dx]
                    if mask is not None:
                        t = jnp.where(mask, t, jnp.bfloat16(0))
                    parts.append(t)
            patch = jnp.concatenate(parts, axis=0)
            acc = jnp.dot(w_ref[...], patch,
                          preferred_element_type=jnp.float32)
            conv_ref[b] = acc.astype(jnp.bfloat16)
            s1 = jnp.sum(acc, axis=1, keepdims=True)
            s2 = jnp.sum(acc * acc, axis=1, keepdims=True)
            stats_ref[b] = jnp.concatenate(
                [s1, s2, jnp.zeros((C, 6), jnp.float32)], axis=1)
    return _conv_stats_kernel


def _bn_res_kernel(conv_ref, x_ref, sc_ref, sh_ref, out_ref):
    out_ref[...] = (conv_ref[...].astype(jnp.float32) * sc_ref[...]
                    + sh_ref[...] + x_ref[...])


@jax.jit
def _residual_block_opt(x_nchw, w_oihw, gamma, beta):
    N, C, H, W = x_nchw.shape
    M = H * W
    xm = x_nchw.reshape(N, C, M)

    wk = jnp.transpose(w_oihw, (0, 2, 3, 1)).reshape(C, 9 * C).astype(jnp.bfloat16)

    B = 4 if N % 4 == 0 else 1
    G = N // B
    cparams = pltpu.CompilerParams(
        dimension_semantics=("parallel",),
        vmem_limit_bytes=64 * 1024 * 1024)

    conv_out, stats = pl.pallas_call(
        _make_conv_stats_kernel(B, H, W),
        grid=(G,),
        in_specs=[
            pl.BlockSpec((B, C, M), lambda i: (i, 0, 0)),
            pl.BlockSpec((C, 9 * C), lambda i: (0, 0)),
        ],
        out_specs=(
            pl.BlockSpec((B, C, M), lambda i: (i, 0, 0)),
            pl.BlockSpec((B, C, 8), lambda i: (i, 0, 0)),
        ),
        out_shape=(
            jax.ShapeDtypeStruct((N, C, M), jnp.bfloat16),
            jax.ShapeDtypeStruct((N, C, 8), jnp.float32),
        ),
        compiler_params=cparams,
    )(xm, wk)

    Mtot = N * M
    mean = jnp.sum(stats[:, :, 0], axis=0) / Mtot
    var = jnp.maximum(jnp.sum(stats[:, :, 1], axis=0) / Mtot - mean * mean, 0.0)
    scale = (gamma.astype(jnp.float32) * jax.lax.rsqrt(var + _BN_EPS))
    shift = beta.astype(jnp.float32) - mean * scale
    scale = scale.reshape(C, 1)
    shift = shift.reshape(C, 1)

    out = pl.pallas_call(
        _bn_res_kernel,
        grid=(G,),
        in_specs=[
            pl.BlockSpec((B, C, M), lambda i: (i, 0, 0)),
            pl.BlockSpec((B, C, M), lambda i: (i, 0, 0)),
            pl.BlockSpec((C, 1), lambda i: (0, 0)),
            pl.BlockSpec((C, 1), lambda i: (0, 0)),
        ],
        out_specs=pl.BlockSpec((B, C, M), lambda i: (i, 0, 0)),
        out_shape=jax.ShapeDtypeStruct((N, C, M), x_nchw.dtype),
        compiler_params=cparams,
    )(conv_out, xm, scale, shift)

    return out.reshape(N, C, H, W)


def kernel(x_nchw, w_oihw, bias, gamma, beta):
    del bias
    return _residual_block_opt(x_nchw, w_oihw, gamma, beta)


# speedup vs baseline: 1.0202x; 1.0202x over previous
import jax
import jax.numpy as jnp
from jax.experimental import pallas as pl
from jax.experimental.pallas import tpu as pltpu

_BN_EPS = 1e-5


def _make_conv_stats_kernel(B, H, W):
    def _conv_stats_kernel(x_ref, w_ref, conv_ref, xb_ref, stats_ref):
        _, C, M = x_ref.shape

        m = jax.lax.broadcasted_iota(jnp.int32, (1, M), 1)
        col = jax.lax.rem(m, W)
        row_ok = {-1: m >= W, 0: None, 1: m < M - W}
        col_ok = {-1: col > 0, 0: None, 1: col < W - 1}

        for b in range(B):
            x = x_ref[b].astype(jnp.bfloat16)
            xb_ref[b] = x
            parts = []
            for ky in range(3):
                dy = ky - 1
                for kx in range(3):
                    dx = kx - 1
                    s = dy * W + dx
                    t = x if s == 0 else jnp.roll(x, -s, axis=1)
                    mask = None
                    if row_ok[dy] is not None and col_ok[dx] is not None:
                        mask = jnp.logical_and(row_ok[dy], col_ok[dx])
                    elif row_ok[dy] is not None:
                        mask = row_ok[dy]
                    elif col_ok[dx] is not None:
                        mask = col_ok[dx]
                    if mask is not None:
                        t = jnp.where(mask, t, jnp.bfloat16(0))
                    parts.append(t)
            patch = jnp.concatenate(parts, axis=0)
            acc = jnp.dot(w_ref[...], patch,
                          preferred_element_type=jnp.float32)
            conv_ref[b] = acc.astype(jnp.bfloat16)
            s1 = jnp.sum(acc, axis=1, keepdims=True)
            s2 = jnp.sum(acc * acc, axis=1, keepdims=True)
            stats_ref[b] = jnp.concatenate(
                [s1, s2, jnp.zeros((C, 6), jnp.float32)], axis=1)
    return _conv_stats_kernel


def _bn_res_kernel(conv_ref, x_ref, sc_ref, sh_ref, out_ref):
    out_ref[...] = (conv_ref[...].astype(jnp.float32) * sc_ref[...]
                    + sh_ref[...] + x_ref[...].astype(jnp.float32))


@jax.jit
def _residual_block_opt(x_nchw, w_oihw, gamma, beta):
    N, C, H, W = x_nchw.shape
    M = H * W
    xm = x_nchw.reshape(N, C, M)

    wk = jnp.transpose(w_oihw, (0, 2, 3, 1)).reshape(C, 9 * C).astype(jnp.bfloat16)

    B = 8 if N % 8 == 0 else 1
    G = N // B
    cparams = pltpu.CompilerParams(
        dimension_semantics=("parallel",),
        vmem_limit_bytes=64 * 1024 * 1024)

    conv_out, xb, stats = pl.pallas_call(
        _make_conv_stats_kernel(B, H, W),
        grid=(G,),
        in_specs=[
            pl.BlockSpec((B, C, M), lambda i: (i, 0, 0)),
            pl.BlockSpec((C, 9 * C), lambda i: (0, 0)),
        ],
        out_specs=(
            pl.BlockSpec((B, C, M), lambda i: (i, 0, 0)),
            pl.BlockSpec((B, C, M), lambda i: (i, 0, 0)),
            pl.BlockSpec((B, C, 8), lambda i: (i, 0, 0)),
        ),
        out_shape=(
            jax.ShapeDtypeStruct((N, C, M), jnp.bfloat16),
            jax.ShapeDtypeStruct((N, C, M), jnp.bfloat16),
            jax.ShapeDtypeStruct((N, C, 8), jnp.float32),
        ),
        compiler_params=cparams,
    )(xm, wk)

    Mtot = N * M
    mean = jnp.sum(stats[:, :, 0], axis=0) / Mtot
    var = jnp.maximum(jnp.sum(stats[:, :, 1], axis=0) / Mtot - mean * mean, 0.0)
    scale = (gamma.astype(jnp.float32) * jax.lax.rsqrt(var + _BN_EPS))
    shift = beta.astype(jnp.float32) - mean * scale
    scale = scale.reshape(C, 1)
    shift = shift.reshape(C, 1)

    out = pl.pallas_call(
        _bn_res_kernel,
        grid=(G,),
        in_specs=[
            pl.BlockSpec((B, C, M), lambda i: (i, 0, 0)),
            pl.BlockSpec((B, C, M), lambda i: (i, 0, 0)),
            pl.BlockSpec((C, 1), lambda i: (0, 0)),
            pl.BlockSpec((C, 1), lambda i: (0, 0)),
        ],
        out_specs=pl.BlockSpec((B, C, M), lambda i: (i, 0, 0)),
        out_shape=jax.ShapeDtypeStruct((N, C, M), x_nchw.dtype),
        compiler_params=cparams,
    )(conv_out, xb, scale, shift)

    return out.reshape(N, C, H, W)


def kernel(x_nchw, w_oihw, bias, gamma, beta):
    del bias
    return _residual_block_opt(x_nchw, w_oihw, gamma, beta)


# speedup vs baseline: 1.1951x; 1.1714x over previous
import jax
import jax.numpy as jnp
from jax.experimental import pallas as pl
from jax.experimental.pallas import tpu as pltpu

_BN_EPS = 1e-5


def _make_conv_stats_kernel(B, H, W):
    def _conv_stats_kernel(x_ref, w_ref, conv_ref, xb_ref, stats_ref):
        _, C, M = x_ref.shape

        m = jax.lax.broadcasted_iota(jnp.int32, (1, M), 1)
        col = jax.lax.rem(m, W)
        mask_l = col > 0
        mask_r = col < W - 1
        mask_top = m >= W
        mask_bot = m < M - W

        for b in range(B):
            x = x_ref[b].astype(jnp.bfloat16)
            xb_ref[b] = x
            xl = jnp.where(mask_l, jnp.roll(x, 1, axis=1), jnp.bfloat16(0))
            xr = jnp.where(mask_r, jnp.roll(x, -1, axis=1), jnp.bfloat16(0))
            x3 = jnp.concatenate([xl, x, xr], axis=0)
            t1 = jnp.dot(w_ref[:, 3 * C:6 * C], x3,
                         preferred_element_type=jnp.float32)
            t0 = jnp.dot(w_ref[:, 0:3 * C], x3,
                         preferred_element_type=jnp.float32)
            t2 = jnp.dot(w_ref[:, 6 * C:9 * C], x3,
                         preferred_element_type=jnp.float32)
            acc = (t1
                   + jnp.where(mask_top, jnp.roll(t0, W, axis=1), 0.0)
                   + jnp.where(mask_bot, jnp.roll(t2, -W, axis=1), 0.0))
            conv_ref[b] = acc.astype(jnp.bfloat16)
            s1 = jnp.sum(acc, axis=1, keepdims=True)
            s2 = jnp.sum(acc * acc, axis=1, keepdims=True)
            stats_ref[b] = jnp.concatenate(
                [s1, s2, jnp.zeros((C, 6), jnp.float32)], axis=1)
    return _conv_stats_kernel


def _bn_res_kernel(conv_ref, x_ref, sc_ref, sh_ref, out_ref):
    out_ref[...] = (conv_ref[...].astype(jnp.float32) * sc_ref[...]
                    + sh_ref[...] + x_ref[...].astype(jnp.float32))


@jax.jit
def _residual_block_opt(x_nchw, w_oihw, gamma, beta):
    N, C, H, W = x_nchw.shape
    M = H * W
    xm = x_nchw.reshape(N, C, M)

    wk = jnp.transpose(w_oihw, (0, 2, 3, 1)).reshape(C, 9 * C).astype(jnp.bfloat16)

    B = 8 if N % 8 == 0 else 1
    G = N // B
    cparams = pltpu.CompilerParams(
        dimension_semantics=("parallel",),
        vmem_limit_bytes=64 * 1024 * 1024)

    conv_out, xb, stats = pl.pallas_call(
        _make_conv_stats_kernel(B, H, W),
        grid=(G,),
        in_specs=[
            pl.BlockSpec((B, C, M), lambda i: (i, 0, 0)),
            pl.BlockSpec((C, 9 * C), lambda i: (0, 0)),
        ],
        out_specs=(
            pl.BlockSpec((B, C, M), lambda i: (i, 0, 0)),
            pl.BlockSpec((B, C, M), lambda i: (i, 0, 0)),
            pl.BlockSpec((B, C, 8), lambda i: (i, 0, 0)),
        ),
        out_shape=(
            jax.ShapeDtypeStruct((N, C, M), jnp.bfloat16),
            jax.ShapeDtypeStruct((N, C, M), jnp.bfloat16),
            jax.ShapeDtypeStruct((N, C, 8), jnp.float32),
        ),
        compiler_params=cparams,
    )(xm, wk)

    Mtot = N * M
    mean = jnp.sum(stats[:, :, 0], axis=0) / Mtot
    var = jnp.maximum(jnp.sum(stats[:, :, 1], axis=0) / Mtot - mean * mean, 0.0)
    scale = (gamma.astype(jnp.float32) * jax.lax.rsqrt(var + _BN_EPS))
    shift = beta.astype(jnp.float32) - mean * scale
    scale = scale.reshape(C, 1)
    shift = shift.reshape(C, 1)

    out = pl.pallas_call(
        _bn_res_kernel,
        grid=(G,),
        in_specs=[
            pl.BlockSpec((B, C, M), lambda i: (i, 0, 0)),
            pl.BlockSpec((B, C, M), lambda i: (i, 0, 0)),
            pl.BlockSpec((C, 1), lambda i: (0, 0)),
            pl.BlockSpec((C, 1), lambda i: (0, 0)),
        ],
        out_specs=pl.BlockSpec((B, C, M), lambda i: (i, 0, 0)),
        out_shape=jax.ShapeDtypeStruct((N, C, M), x_nchw.dtype),
        compiler_params=cparams,
    )(conv_out, xb, scale, shift)

    return out.reshape(N, C, H, W)


def kernel(x_nchw, w_oihw, bias, gamma, beta):
    del bias
    return _residual_block_opt(x_nchw, w_oihw, gamma, beta)


# speedup vs baseline: 2.5491x; 2.1329x over previous
import jax
import jax.numpy as jnp
from jax.experimental import pallas as pl
from jax.experimental.pallas import tpu as pltpu

_BN_EPS = 1e-5


def _make_fused_kernel(B, G, H, W, N):
    M = H * W

    def _fused(x_ref, w_ref, gb_ref, out_ref, conv_v, stats_v, ss_v):
        C = x_ref.shape[2]
        i = pl.program_id(0)
        phase0 = i < G

        m = jax.lax.broadcasted_iota(jnp.int32, (M, 1), 0)
        col = jax.lax.rem(m, W)
        masks = {
            -1: col > 0,
            0: None,
            1: col < W - 1,
        }
        rowm = {
            -1: m >= W,
            0: None,
            1: m < M - W,
        }

        @pl.when(phase0)
        def _():
            @pl.when(i == 0)
            def _():
                stats_v[...] = jnp.zeros((8, C), jnp.float32)

            st = stats_v[...]
            for b in range(B):
                x = x_ref[b].astype(jnp.bfloat16)
                parts = []
                for ky in range(3):
                    dy = ky - 1
                    for kx in range(3):
                        dx = kx - 1
                        s = dy * W + dx
                        t = x if s == 0 else jnp.roll(x, -s, axis=0)
                        mask = None
                        if rowm[dy] is not None and masks[dx] is not None:
                            mask = jnp.logical_and(rowm[dy], masks[dx])
                        elif rowm[dy] is not None:
                            mask = rowm[dy]
                        elif masks[dx] is not None:
                            mask = masks[dx]
                        if mask is not None:
                            t = jnp.where(mask, t, jnp.bfloat16(0))
                        parts.append(t)
                patch = jnp.concatenate(parts, axis=1)
                acc = jnp.dot(patch, w_ref[...],
                              preferred_element_type=jnp.float32)
                conv_v[i * B + b] = acc.astype(jnp.bfloat16)
                s1 = jnp.sum(acc, axis=0, keepdims=True)
                s2 = jnp.sum(acc * acc, axis=0, keepdims=True)
                st = st + jnp.concatenate(
                    [s1, s2, jnp.zeros((6, C), jnp.float32)], axis=0)
            stats_v[...] = st

        @pl.when(jnp.logical_not(phase0))
        def _():
            @pl.when(i == G)
            def _():
                st = stats_v[...]
                mean = st[0:1, :] / (N * M)
                var = jnp.maximum(st[1:2, :] / (N * M) - mean * mean, 0.0)
                scale = gb_ref[0:1, :] * jax.lax.rsqrt(var + _BN_EPS)
                shift = gb_ref[1:2, :] - mean * scale
                ss_v[...] = jnp.concatenate(
                    [scale, shift, jnp.zeros((6, C), jnp.float32)], axis=0)

            g = i - G
            scale = ss_v[0:1, :]
            shift = ss_v[1:2, :]
            for b in range(B):
                out_ref[b] = (conv_v[g * B + b].astype(jnp.float32) * scale
                              + shift + x_ref[b])

    return _fused


@jax.jit
def _residual_block_opt(x_nchw, w_oihw, gamma, beta):
    N, C, H, W = x_nchw.shape
    M = H * W
    xm = jnp.transpose(x_nchw, (0, 2, 3, 1)).reshape(N, M, C)

    wk = jnp.transpose(w_oihw, (2, 3, 1, 0)).reshape(9 * C, C).astype(jnp.bfloat16)
    gb = jnp.concatenate(
        [gamma.reshape(1, C).astype(jnp.float32),
         beta.reshape(1, C).astype(jnp.float32),
         jnp.zeros((6, C), jnp.float32)], axis=0)

    B = 8 if N % 8 == 0 else 1
    G = N // B

    out = pl.pallas_call(
        _make_fused_kernel(B, G, H, W, N),
        grid=(2 * G,),
        in_specs=[
            pl.BlockSpec((B, M, C), lambda i: (jnp.where(i < G, i, i - G), 0, 0)),
            pl.BlockSpec((9 * C, C), lambda i: (0, 0)),
            pl.BlockSpec((8, C), lambda i: (0, 0)),
        ],
        out_specs=pl.BlockSpec(
            (B, M, C), lambda i: (jnp.where(i < G, 0, i - G), 0, 0)),
        out_shape=jax.ShapeDtypeStruct((N, M, C), x_nchw.dtype),
        scratch_shapes=[
            pltpu.VMEM((N, M, C), jnp.bfloat16),
            pltpu.VMEM((8, C), jnp.float32),
            pltpu.VMEM((8, C), jnp.float32),
        ],
        compiler_params=pltpu.CompilerParams(
            dimension_semantics=("arbitrary",),
            vmem_limit_bytes=56 * 1024 * 1024),
    )(xm, wk, gb)

    return jnp.transpose(out.reshape(N, H, W, C), (0, 3, 1, 2))


def kernel(x_nchw, w_oihw, bias, gamma, beta):
    del bias
    return _residual_block_opt(x_nchw, w_oihw, gamma, beta)
